# baseline (speedup 1.0000x reference)
import functools
import jax
import jax.numpy as jnp
from jax import lax
from jax.experimental import pallas as pl
from jax.experimental.pallas import tpu as pltpu
from jax.experimental.pallas import tpu_sc as plsc

B, H, W, EMB = 1024, 25, 25, 32
HW = H * W
NC, NS, L = 2, 16, 16
NW = NC * NS
P_PER_W = 20
PB = 4
NB = P_PER_W // PB
NCHUNK = B // L


def _sc_body(i0, i1, i2, i3, i4, i5, tab_hbm, out_hbm,
             tab_v, t1_v, t2_v, idxa_v, idxb_v, acc0_v, acc1_v,
             semi_a, semi_b, sem0, sem1):
    wid = lax.axis_index("s") * NC + lax.axis_index("c")
    idx_refs = (i0, i1, i2, i3, i4, i5)
    start = wid * P_PER_W - jnp.minimum(wid, NW * P_PER_W - HW)

    pltpu.sync_copy(tab_hbm, tab_v)
    iota = lax.iota(jnp.int32, L)

    def build_t1(e, _):
        e_vec = jnp.full((L,), 0, jnp.int32) + e
        for jc in range(4):
            j = jnp.minimum(iota + (jc * L), 62)
            c0 = (j * 391) >> 13
            r = j - c0 * 21
            c1 = (r * 11) >> 5
            c2 = r - c1 * 3
            v = (plsc.load_gather(tab_v, [c0, e_vec])
                 + plsc.load_gather(tab_v, [c1 + 3, e_vec])
                 + plsc.load_gather(tab_v, [c2 + 10, e_vec]))
            t1_v[e, pl.ds(jc * L, L)] = v
        return _

    def build_t2(e, _):
        e_vec = jnp.full((L,), 0, jnp.int32) + e
        for jc in range(7):
            j = jnp.minimum(iota + (jc * L), 107)
            c3 = (j * 57) >> 11
            r = j - c3 * 36
            c4 = (r * 43) >> 8
            c5 = r - c4 * 6
            v = (plsc.load_gather(tab_v, [c3 + 13, e_vec])
                 + plsc.load_gather(tab_v, [c4 + 16, e_vec])
                 + plsc.load_gather(tab_v, [c5 + 22, e_vec]))
            t2_v[e, pl.ds(jc * L, L)] = v
        return _

    lax.fori_loop(0, EMB, build_t1, 0)
    lax.fori_loop(0, EMB, build_t2, 0)

    def issue_idx(g, buf_v, sem):
        boff = jnp.minimum(start + g * PB, HW - PB) * B
        for p in range(6):
            pltpu.async_copy(idx_refs[p].at[pl.ds(boff, PB * B)],
                             buf_v.at[p], sem)

    def wait_idx(buf_v, sem):
        for p in range(6):
            pltpu.make_async_copy(idx_refs[p].at[pl.ds(0, PB * B)],
                                  buf_v.at[p], sem).wait()

    def compute_pos(acc_v, idx_v, k):
        @plsc.parallel_loop(0, NCHUNK, step=1, unroll=4)
        def chunk_body(c):
            s = k * B + c * L
            v0 = idx_v[0, pl.ds(s, L)]
            v1 = idx_v[1, pl.ds(s, L)]
            v2 = idx_v[2, pl.ds(s, L)]
            v3 = idx_v[3, pl.ds(s, L)]
            v4 = idx_v[4, pl.ds(s, L)]
            v5 = idx_v[5, pl.ds(s, L)]
            j1 = (v0 * 7 + v1) * 3 + v2
            j2 = (v3 * 6 + v4) * 6 + v5
            d = (c >> 3) * 1024 + (c & 7) * L
            for e0 in range(0, EMB, 16):
                g1 = [plsc.load_gather(t1_v.at[e0 + k2], [j1])
                      for k2 in range(16)]
                g2 = [plsc.load_gather(t2_v.at[e0 + k2], [j2])
                      for k2 in range(16)]
                for k2 in range(16):
                    e = e0 + k2
                    eo = (e // 8) * 8192 + (e % 8) * 128
                    acc_v[pl.ds(d + eo, L)] = g1[k2] + g2[k2]

    issue_idx(0, idxa_v, semi_a)

    def batch_body(g, _):
        even = lax.rem(g, 2) == 0

        def run_batch(idx_v, sem_cur, idx_nxt, sem_nxt):
            wait_idx(idx_v, sem_cur)

            @pl.when(g < NB - 1)
            def _pf():
                issue_idx(g + 1, idx_nxt, sem_nxt)

            for k in range(PB):
                pos = jnp.minimum(start + g * PB, HW - PB) + k
                h = (pos * 1311) >> 15
                w = pos - h * 25
                acc_v, sem = (acc0_v, sem0) if k % 2 == 0 else (acc1_v, sem1)

                @pl.when((g > 0) | (k >= 2))
                def _drain():
                    pltpu.make_async_copy(acc_v, out_hbm.at[h, w], sem).wait()
                compute_pos(acc_v, idx_v, k)
                pltpu.async_copy(acc_v, out_hbm.at[h, w], sem)

        @pl.when(even)
        def _a():
            run_batch(idxa_v, semi_a, idxb_v, semi_b)

        @pl.when(~even)
        def _b():
            run_batch(idxb_v, semi_b, idxa_v, semi_a)
        return _

    lax.fori_loop(0, NB, batch_body, 0)
    lastp = start + P_PER_W - 1
    lh = (lastp * 1311) >> 15
    lw = lastp - lh * 25
    pltpu.make_async_copy(acc0_v, out_hbm.at[lh, lw], sem0).wait()
    pltpu.make_async_copy(acc1_v, out_hbm.at[lh, lw], sem1).wait()


@jax.jit
def _run(i0, i1, i2, i3, i4, i5, tab):
    mesh = plsc.VectorSubcoreMesh(core_axis_name="c", subcore_axis_name="s")
    f = pl.kernel(
        _sc_body,
        out_type=jax.ShapeDtypeStruct((H, W, EMB * B), jnp.float32),
        mesh=mesh,
        compiler_params=pltpu.CompilerParams(
            use_tc_tiling_on_sc=False, needs_layout_passes=False),
        scratch_types=[
            pltpu.VMEM((28, EMB), jnp.float32),
            pltpu.VMEM((EMB, 64), jnp.float32),
            pltpu.VMEM((EMB, 112), jnp.float32),
            pltpu.VMEM((6, PB * B), jnp.int32),
            pltpu.VMEM((6, PB * B), jnp.int32),
            pltpu.VMEM((EMB * B,), jnp.float32),
            pltpu.VMEM((EMB * B,), jnp.float32),
            pltpu.SemaphoreType.DMA,
            pltpu.SemaphoreType.DMA,
            pltpu.SemaphoreType.DMA,
            pltpu.SemaphoreType.DMA,
        ],
    )
    return f(i0, i1, i2, i3, i4, i5, tab)


def kernel(card_counts, card_colors, card_shapes, card_selections,
           leader_rotations, follower_rotations, embedding_table):
    tf = lambda x: jnp.transpose(x, (1, 2, 0)).reshape(-1).astype(jnp.int32)
    out = _run(tf(card_counts), tf(card_colors), tf(card_shapes),
               tf(card_selections), tf(leader_rotations),
               tf(follower_rotations), embedding_table.astype(jnp.float32))
    o6 = out.reshape(H, W, EMB // 8, B // 128, 8, 128)
    return jnp.transpose(o6, (3, 5, 2, 4, 0, 1)).reshape(B, EMB, H, W)


# speedup vs baseline: 1.0710x; 1.0710x over previous
import functools
import jax
import jax.numpy as jnp
from jax import lax
from jax.experimental import pallas as pl
from jax.experimental.pallas import tpu as pltpu
from jax.experimental.pallas import tpu_sc as plsc

B, H, W, EMB = 1024, 25, 25, 32
HW = H * W
NC, NS, L = 2, 16, 16
NW = NC * NS
P_PER_W = 20
PB = 4
NB = P_PER_W // PB
NCHUNK = B // L


def _sc_body(i0, i1, i2, i3, i4, i5, tab_hbm, out_hbm,
             tab_v, t1_v, t2_v, idxa_v, idxb_v, acc0_v, acc1_v,
             semi_a, semi_b, sem0, sem1):
    wid = lax.axis_index("s") * NC + lax.axis_index("c")
    idx_refs = (i0, i1, i2, i3, i4, i5)
    start = wid * P_PER_W - jnp.minimum(wid, NW * P_PER_W - HW)

    pltpu.sync_copy(tab_hbm, tab_v)
    iota = lax.iota(jnp.int32, L)

    def build_t1(e, _):
        e_vec = jnp.full((L,), 0, jnp.int32) + e
        for jc in range(4):
            j = jnp.minimum(iota + (jc * L), 62)
            c0 = (j * 391) >> 13
            r = j - c0 * 21
            c1 = (r * 11) >> 5
            c2 = r - c1 * 3
            v = (plsc.load_gather(tab_v, [c0, e_vec])
                 + plsc.load_gather(tab_v, [c1 + 3, e_vec])
                 + plsc.load_gather(tab_v, [c2 + 10, e_vec]))
            t1_v[e, pl.ds(jc * L, L)] = v
        return _

    def build_t2(e, _):
        e_vec = jnp.full((L,), 0, jnp.int32) + e
        for jc in range(7):
            j = jnp.minimum(iota + (jc * L), 107)
            c3 = (j * 57) >> 11
            r = j - c3 * 36
            c4 = (r * 43) >> 8
            c5 = r - c4 * 6
            v = (plsc.load_gather(tab_v, [c3 + 13, e_vec])
                 + plsc.load_gather(tab_v, [c4 + 16, e_vec])
                 + plsc.load_gather(tab_v, [c5 + 22, e_vec]))
            t2_v[e, pl.ds(jc * L, L)] = v
        return _

    lax.fori_loop(0, EMB, build_t1, 0)
    lax.fori_loop(0, EMB, build_t2, 0)

    def issue_idx(g, buf_v, sem):
        boff = jnp.minimum(start + g * PB, HW - PB) * B
        for p in range(6):
            pltpu.async_copy(idx_refs[p].at[pl.ds(boff, PB * B)],
                             buf_v.at[p], sem)

    def wait_idx(buf_v, sem):
        for p in range(6):
            pltpu.make_async_copy(idx_refs[p].at[pl.ds(0, PB * B)],
                                  buf_v.at[p], sem).wait()

    def compute_pos(acc_v, idx_v, k):
        @plsc.parallel_loop(0, NCHUNK, step=1, unroll=4)
        def chunk_body(c):
            s = k * B + c * L
            v0 = idx_v[0, pl.ds(s, L)]
            v1 = idx_v[1, pl.ds(s, L)]
            v2 = idx_v[2, pl.ds(s, L)]
            v3 = idx_v[3, pl.ds(s, L)]
            v4 = idx_v[4, pl.ds(s, L)]
            v5 = idx_v[5, pl.ds(s, L)]
            j1 = (v0 * 7 + v1) * 3 + v2
            j2 = (v3 * 6 + v4) * 6 + v5
            d = (c >> 3) * 1024 + (c & 7) * L
            for e0 in range(0, EMB, 8):
                g1 = [plsc.load_gather(t1_v.at[e0 + k2], [j1])
                      for k2 in range(8)]
                g2 = [plsc.load_gather(t2_v.at[e0 + k2], [j2])
                      for k2 in range(8)]
                for k2 in range(8):
                    e = e0 + k2
                    eo = (e // 8) * 8192 + (e % 8) * 128
                    acc_v[pl.ds(d + eo, L)] = g1[k2] + g2[k2]

    issue_idx(0, idxa_v, semi_a)

    def batch_body(g, _):
        even = lax.rem(g, 2) == 0

        def run_batch(idx_v, sem_cur, idx_nxt, sem_nxt):
            wait_idx(idx_v, sem_cur)

            @pl.when(g < NB - 1)
            def _pf():
                issue_idx(g + 1, idx_nxt, sem_nxt)

            for k in range(PB):
                pos = jnp.minimum(start + g * PB, HW - PB) + k
                h = (pos * 1311) >> 15
                w = pos - h * 25
                acc_v, sem = (acc0_v, sem0) if k % 2 == 0 else (acc1_v, sem1)

                @pl.when((g > 0) | (k >= 2))
                def _drain():
                    pltpu.make_async_copy(acc_v, out_hbm.at[h, w], sem).wait()
                compute_pos(acc_v, idx_v, k)
                pltpu.async_copy(acc_v, out_hbm.at[h, w], sem)

        @pl.when(even)
        def _a():
            run_batch(idxa_v, semi_a, idxb_v, semi_b)

        @pl.when(~even)
        def _b():
            run_batch(idxb_v, semi_b, idxa_v, semi_a)
        return _

    lax.fori_loop(0, NB, batch_body, 0)
    lastp = start + P_PER_W - 1
    lh = (lastp * 1311) >> 15
    lw = lastp - lh * 25
    pltpu.make_async_copy(acc0_v, out_hbm.at[lh, lw], sem0).wait()
    pltpu.make_async_copy(acc1_v, out_hbm.at[lh, lw], sem1).wait()


@jax.jit
def _run(i0, i1, i2, i3, i4, i5, tab):
    mesh = plsc.VectorSubcoreMesh(core_axis_name="c", subcore_axis_name="s")
    f = pl.kernel(
        _sc_body,
        out_type=jax.ShapeDtypeStruct((H, W, EMB * B), jnp.float32),
        mesh=mesh,
        compiler_params=pltpu.CompilerParams(
            use_tc_tiling_on_sc=False, needs_layout_passes=False),
        scratch_types=[
            pltpu.VMEM((28, EMB), jnp.float32),
            pltpu.VMEM((EMB, 64), jnp.float32),
            pltpu.VMEM((EMB, 112), jnp.float32),
            pltpu.VMEM((6, PB * B), jnp.int32),
            pltpu.VMEM((6, PB * B), jnp.int32),
            pltpu.VMEM((EMB * B,), jnp.float32),
            pltpu.VMEM((EMB * B,), jnp.float32),
            pltpu.SemaphoreType.DMA,
            pltpu.SemaphoreType.DMA,
            pltpu.SemaphoreType.DMA,
            pltpu.SemaphoreType.DMA,
        ],
    )
    return f(i0, i1, i2, i3, i4, i5, tab)


def kernel(card_counts, card_colors, card_shapes, card_selections,
           leader_rotations, follower_rotations, embedding_table):
    tf = lambda x: jnp.transpose(x, (1, 2, 0)).reshape(-1).astype(jnp.int32)
    out = _run(tf(card_counts), tf(card_colors), tf(card_shapes),
               tf(card_selections), tf(leader_rotations),
               tf(follower_rotations), embedding_table.astype(jnp.float32))
    o6 = out.reshape(H, W, EMB // 8, B // 128, 8, 128)
    return jnp.transpose(o6, (3, 5, 2, 4, 0, 1)).reshape(B, EMB, H, W)
